# Initial kernel scaffold; baseline (speedup 1.0000x reference)
import jax
import jax.numpy as jnp
from jax.experimental import pallas as pl


def kernel(X, edge_index, edge_weight, edge_attr, q, ws1, bs1, ws2, bs2, ws3, bs3, wt):
    raise NotImplementedError("write your pallas kernel here")


# speedup vs baseline: 3.2657x; 3.2657x over previous
import functools
import math

import jax
import jax.numpy as jnp
import numpy as np
from jax.experimental import pallas as pl
from jax.experimental.pallas import tpu as pltpu

N = 10000
E = 160000
H = 128
R = 32
L = 2
CUTOFF_UPPER = 4.5

EB = 2000
NB = 500


def _silu(x):
    return x / (1.0 + jnp.exp(-x))


def _mlp_body(attr_ref, ew_ref, w1_ref, b1_ref, w2_ref, b2_ref, w3_ref, b3_ref,
              out_ref):
    x = _silu(jnp.dot(attr_ref[...], w1_ref[...].T,
                      preferred_element_type=jnp.float32) + b1_ref[...])
    x = _silu(jnp.dot(x, w2_ref[...].T,
                      preferred_element_type=jnp.float32) + b2_ref[...])
    x = _silu(jnp.dot(x, w3_ref[...].T,
                      preferred_element_type=jnp.float32) + b3_ref[...])
    d = ew_ref[...].reshape(-1, 1)
    c = 0.5 * (jnp.cos(d * (math.pi / CUTOFF_UPPER)) + 1.0)
    c = jnp.where(d < CUTOFF_UPPER, c, 0.0)
    out_ref[...] = x * c


def _mlp_call(edge_attr, ew2, w1, b1, w2, b2, w3, b3):
    grid = (E // EB,)
    return pl.pallas_call(
        _mlp_body,
        grid=grid,
        in_specs=[
            pl.BlockSpec((EB, R), lambda i: (i, 0)),
            pl.BlockSpec((1, 1, EB), lambda i: (i, 0, 0)),
            pl.BlockSpec((H, R), lambda i: (0, 0)),
            pl.BlockSpec((1, H), lambda i: (0, 0)),
            pl.BlockSpec((2 * H, H), lambda i: (0, 0)),
            pl.BlockSpec((1, 2 * H), lambda i: (0, 0)),
            pl.BlockSpec((3 * H, 2 * H), lambda i: (0, 0)),
            pl.BlockSpec((1, 3 * H), lambda i: (0, 0)),
        ],
        out_specs=pl.BlockSpec((EB, 3 * H), lambda i: (i, 0)),
        out_shape=jax.ShapeDtypeStruct((E, 3 * H), jnp.float32),
    )(edge_attr, ew2, w1, b1, w2, b2, w3, b3)


def _decompose(t):
    trm = (t[:, 0] + t[:, 4] + t[:, 8]) * (1.0 / 3.0)
    a01 = 0.5 * (t[:, 1] - t[:, 3])
    a02 = 0.5 * (t[:, 2] - t[:, 6])
    a12 = 0.5 * (t[:, 5] - t[:, 7])
    s00 = t[:, 0] - trm
    s01 = 0.5 * (t[:, 1] + t[:, 3])
    s02 = 0.5 * (t[:, 2] + t[:, 6])
    s11 = t[:, 4] - trm
    s12 = 0.5 * (t[:, 5] + t[:, 7])
    return [trm, a01, a02, a12, s00, s01, s02, s11, s12]


def _mix(c, w0, w1, w2):
    out = [jnp.dot(c[0], w0.T, preferred_element_type=jnp.float32)]
    for k in range(1, 4):
        out.append(jnp.dot(c[k], w1.T, preferred_element_type=jnp.float32))
    for k in range(4, 9):
        out.append(jnp.dot(c[k], w2.T, preferred_element_type=jnp.float32))
    return out


def _recon(c):
    i, a01, a02, a12, s00, s01, s02, s11, s12 = c
    return [i + s00, s01 + a01, s02 + a02,
            s01 - a01, i + s11, s12 + a12,
            s02 - a02, s12 - a12, i - s00 - s11]


def _mm33(a, b):
    out = []
    for i in range(3):
        for j in range(3):
            acc = a[3 * i] * b[j]
            for k in range(1, 3):
                acc = acc + a[3 * i + k] * b[3 * k + j]
            out.append(acc)
    return out


def _prep_body(x_ref, w0_ref, w1_ref, w2_ref, xn_ref, feat_ref, y_ref):
    x = x_ref[...]
    norm = jnp.sum(x * x, axis=1)
    xn = x * (1.0 / (norm + 1.0))[:, None, :]
    xn_ref[...] = xn
    c0 = _decompose(xn)
    feat = _mix(c0, w0_ref[...], w1_ref[...], w2_ref[...])
    feat_ref[...] = jnp.stack(feat, axis=1)
    y_ref[...] = jnp.stack(_recon(feat), axis=1)


def _prep_call(X9, w0, w1, w2):
    grid = (N // NB,)
    outs = [jax.ShapeDtypeStruct((N, 9, H), jnp.float32)] * 3
    return pl.pallas_call(
        _prep_body,
        grid=grid,
        in_specs=[
            pl.BlockSpec((NB, 9, H), lambda i: (i, 0, 0)),
            pl.BlockSpec((H, H), lambda i: (0, 0)),
            pl.BlockSpec((H, H), lambda i: (0, 0)),
            pl.BlockSpec((H, H), lambda i: (0, 0)),
        ],
        out_specs=[pl.BlockSpec((NB, 9, H), lambda i: (i, 0, 0))] * 3,
        out_shape=outs,
    )(X9, w0, w1, w2)


def _post_body(m_ref, y_ref, xn_ref, q_ref, w3_ref, w4_ref, w5_ref, out_ref):
    mc = [m_ref[:, k] for k in range(9)]
    m = _recon(mc)
    y = [y_ref[:, k] for k in range(9)]
    a2 = _mm33(m, y)
    b = _mm33(y, m)
    fac = 1.0 + 0.1 * q_ref[...].reshape(-1, 1)
    t = [fac * (a2[k] + b[k]) for k in range(9)]
    c2 = _decompose(jnp.stack(t, axis=1))
    dX = _recon(_mix(c2, w3_ref[...], w4_ref[...], w5_ref[...]))
    dd = _mm33(dX, dX)
    xn = xn_ref[...]
    out = [xn[:, k] + dX[k] + dd[k] for k in range(9)]
    out_ref[...] = jnp.stack(out, axis=1)


def _post_call(msgc9, Y9, Xn9, q2, w3, w4, w5):
    grid = (N // NB,)
    return pl.pallas_call(
        _post_body,
        grid=grid,
        in_specs=[
            pl.BlockSpec((NB, 9, H), lambda i: (i, 0, 0)),
            pl.BlockSpec((NB, 9, H), lambda i: (i, 0, 0)),
            pl.BlockSpec((NB, 9, H), lambda i: (i, 0, 0)),
            pl.BlockSpec((1, 1, NB), lambda i: (i, 0, 0)),
            pl.BlockSpec((H, H), lambda i: (0, 0)),
            pl.BlockSpec((H, H), lambda i: (0, 0)),
            pl.BlockSpec((H, H), lambda i: (0, 0)),
        ],
        out_specs=pl.BlockSpec((NB, 9, H), lambda i: (i, 0, 0)),
        out_shape=jax.ShapeDtypeStruct((N, 9, H), jnp.float32),
    )(msgc9, Y9, Xn9, q2, w3, w4, w5)


def _ws3_perm():
    perm = np.empty((3 * H,), dtype=np.int32)
    for c in range(H // 16):
        for g in range(3):
            for hh in range(16):
                perm[c * 48 + g * 16 + hh] = 3 * (16 * c + hh) + g
    return perm


_PERM = _ws3_perm()


def kernel(X, edge_index, edge_weight, edge_attr, q, ws1, bs1, ws2, bs2, ws3,
           bs3, wt):
    X9 = jnp.transpose(X.reshape(N, H, 9), (0, 2, 1))
    ew2 = edge_weight.reshape(E // EB, 1, EB)
    q2 = q.reshape(N // NB, 1, NB)
    src = edge_index[1]
    dst = edge_index[0]
    for l in range(L):
        w3p = ws3[l][_PERM]
        b3p = bs3[l][_PERM]
        ea = _mlp_call(edge_attr, ew2, ws1[l], bs1[l].reshape(1, H),
                       ws2[l], bs2[l].reshape(1, 2 * H), w3p,
                       b3p.reshape(1, 3 * H))
        Xn9, feat9, Y9 = _prep_call(X9, wt[l, 0], wt[l, 1], wt[l, 2])
        eag = jnp.transpose(ea.reshape(E, 8, 3, 16), (0, 2, 1, 3)).reshape(E, 3, H)
        w9 = jnp.concatenate([eag[:, 0:1], jnp.repeat(eag[:, 1:2], 3, axis=1),
                              jnp.repeat(eag[:, 2:3], 5, axis=1)], axis=1)
        gath = jnp.take(feat9, src, axis=0)
        msgc9 = jnp.zeros((N, 9, H), jnp.float32).at[dst].add(w9 * gath)
        X9 = _post_call(msgc9, Y9, Xn9, q2, wt[l, 3], wt[l, 4], wt[l, 5])
    return jnp.transpose(X9, (0, 2, 1)).reshape(N, H, 3, 3)


# speedup vs baseline: 16.8663x; 5.1646x over previous
import functools
import math

import jax
import jax.numpy as jnp
import numpy as np
from jax import lax
from jax.experimental import pallas as pl
from jax.experimental.pallas import tpu as pltpu
from jax.experimental.pallas import tpu_sc as plsc

N = 10000
E = 160000
H = 128
R = 32
L = 2
CUTOFF_UPPER = 4.5

EB = 2000
NB = 500


def _silu(x):
    return x / (1.0 + jnp.exp(-x))


def _mlp_body(attr_ref, ew_ref, w1_ref, b1_ref, w2_ref, b2_ref, w3_ref, b3_ref,
              out_ref):
    x = _silu(jnp.dot(attr_ref[...], w1_ref[...].T,
                      preferred_element_type=jnp.float32) + b1_ref[...])
    x = _silu(jnp.dot(x, w2_ref[...].T,
                      preferred_element_type=jnp.float32) + b2_ref[...])
    x = _silu(jnp.dot(x, w3_ref[...].T,
                      preferred_element_type=jnp.float32) + b3_ref[...])
    d = ew_ref[...].reshape(-1, 1)
    c = 0.5 * (jnp.cos(d * (math.pi / CUTOFF_UPPER)) + 1.0)
    c = jnp.where(d < CUTOFF_UPPER, c, 0.0)
    out_ref[...] = x * c


def _mlp_call(edge_attr, ew2, w1, b1, w2, b2, w3, b3):
    grid = (E // EB,)
    return pl.pallas_call(
        _mlp_body,
        grid=grid,
        in_specs=[
            pl.BlockSpec((EB, R), lambda i: (i, 0)),
            pl.BlockSpec((1, 1, EB), lambda i: (i, 0, 0)),
            pl.BlockSpec((H, R), lambda i: (0, 0)),
            pl.BlockSpec((1, H), lambda i: (0, 0)),
            pl.BlockSpec((2 * H, H), lambda i: (0, 0)),
            pl.BlockSpec((1, 2 * H), lambda i: (0, 0)),
            pl.BlockSpec((3 * H, 2 * H), lambda i: (0, 0)),
            pl.BlockSpec((1, 3 * H), lambda i: (0, 0)),
        ],
        out_specs=pl.BlockSpec((EB, 3 * H), lambda i: (i, 0)),
        out_shape=jax.ShapeDtypeStruct((E, 3 * H), jnp.float32),
    )(edge_attr, ew2, w1, b1, w2, b2, w3, b3)


def _decompose(t):
    trm = (t[:, 0] + t[:, 4] + t[:, 8]) * (1.0 / 3.0)
    a01 = 0.5 * (t[:, 1] - t[:, 3])
    a02 = 0.5 * (t[:, 2] - t[:, 6])
    a12 = 0.5 * (t[:, 5] - t[:, 7])
    s00 = t[:, 0] - trm
    s01 = 0.5 * (t[:, 1] + t[:, 3])
    s02 = 0.5 * (t[:, 2] + t[:, 6])
    s11 = t[:, 4] - trm
    s12 = 0.5 * (t[:, 5] + t[:, 7])
    return [trm, a01, a02, a12, s00, s01, s02, s11, s12]


def _mix(c, w0, w1, w2):
    out = [jnp.dot(c[0], w0.T, preferred_element_type=jnp.float32)]
    for k in range(1, 4):
        out.append(jnp.dot(c[k], w1.T, preferred_element_type=jnp.float32))
    for k in range(4, 9):
        out.append(jnp.dot(c[k], w2.T, preferred_element_type=jnp.float32))
    return out


def _recon(c):
    i, a01, a02, a12, s00, s01, s02, s11, s12 = c
    return [i + s00, s01 + a01, s02 + a02,
            s01 - a01, i + s11, s12 + a12,
            s02 - a02, s12 - a12, i - s00 - s11]


def _mm33(a, b):
    out = []
    for i in range(3):
        for j in range(3):
            acc = a[3 * i] * b[j]
            for k in range(1, 3):
                acc = acc + a[3 * i + k] * b[3 * k + j]
            out.append(acc)
    return out


def _prep_body(x_ref, w0_ref, w1_ref, w2_ref, xn_ref, feat_ref, y_ref):
    x = x_ref[...]
    norm = jnp.sum(x * x, axis=1)
    xn = x * (1.0 / (norm + 1.0))[:, None, :]
    xn_ref[...] = xn
    c0 = _decompose(xn)
    feat = _mix(c0, w0_ref[...], w1_ref[...], w2_ref[...])
    feat_ref[...] = jnp.stack(feat, axis=1)
    y_ref[...] = jnp.stack(_recon(feat), axis=1)


def _prep_call(X9, w0, w1, w2):
    grid = (N // NB,)
    outs = [jax.ShapeDtypeStruct((N, 9, H), jnp.float32)] * 3
    return pl.pallas_call(
        _prep_body,
        grid=grid,
        in_specs=[
            pl.BlockSpec((NB, 9, H), lambda i: (i, 0, 0)),
            pl.BlockSpec((H, H), lambda i: (0, 0)),
            pl.BlockSpec((H, H), lambda i: (0, 0)),
            pl.BlockSpec((H, H), lambda i: (0, 0)),
        ],
        out_specs=[pl.BlockSpec((NB, 9, H), lambda i: (i, 0, 0))] * 3,
        out_shape=outs,
    )(X9, w0, w1, w2)


def _post_body(m_ref, y_ref, xn_ref, q_ref, w3_ref, w4_ref, w5_ref, out_ref):
    mc = [m_ref[:, k] for k in range(9)]
    m = _recon(mc)
    y = [y_ref[:, k] for k in range(9)]
    a2 = _mm33(m, y)
    b = _mm33(y, m)
    fac = 1.0 + 0.1 * q_ref[...].reshape(-1, 1)
    t = [fac * (a2[k] + b[k]) for k in range(9)]
    c2 = _decompose(jnp.stack(t, axis=1))
    dX = _recon(_mix(c2, w3_ref[...], w4_ref[...], w5_ref[...]))
    dd = _mm33(dX, dX)
    xn = xn_ref[...]
    out = [xn[:, k] + dX[k] + dd[k] for k in range(9)]
    out_ref[...] = jnp.stack(out, axis=1)


def _post_call(msgc9, Y9, Xn9, q2, w3, w4, w5):
    grid = (N // NB,)
    return pl.pallas_call(
        _post_body,
        grid=grid,
        in_specs=[
            pl.BlockSpec((NB, 9, H), lambda i: (i, 0, 0)),
            pl.BlockSpec((NB, 9, H), lambda i: (i, 0, 0)),
            pl.BlockSpec((NB, 9, H), lambda i: (i, 0, 0)),
            pl.BlockSpec((1, 1, NB), lambda i: (i, 0, 0)),
            pl.BlockSpec((H, H), lambda i: (0, 0)),
            pl.BlockSpec((H, H), lambda i: (0, 0)),
            pl.BlockSpec((H, H), lambda i: (0, 0)),
        ],
        out_specs=pl.BlockSpec((NB, 9, H), lambda i: (i, 0, 0)),
        out_shape=jax.ShapeDtypeStruct((N, 9, H), jnp.float32),
    )(msgc9, Y9, Xn9, q2, w3, w4, w5)


CB = 80
ET = E // 16
NBATCH = ET // CB
NPAD = 10240
NT = NPAD // 16
ZR = NT // 5
F = 144


def _sc_msg_body(featc, eac, srcc, dst2d, out, acc, src_v, dst_v,
                 rows_v, w_v, gsem):
    core = lax.axis_index("c")
    sub = lax.axis_index("s")
    z16 = jnp.zeros((16,), jnp.float32)
    pltpu.sync_copy(dst2d.at[sub], dst_v)

    for cc in range(4):
        chunk = core * 4 + cc

        def zrow(i, carry):
            for p in range(9):
                rows_v[i, pl.ds(16 * p, 16)] = z16
            return carry

        lax.fori_loop(0, CB, zrow, 0)
        for z in range(NT // CB):
            pltpu.sync_copy(rows_v, acc.at[pl.ds(sub * NT + z * CB, CB)])
        plsc.subcore_barrier()
        base = chunk * E + sub * ET
        pltpu.sync_copy(srcc.at[pl.ds(base, ET)], src_v)

        def batch_body(b, carry):
            pltpu.async_copy(featc.at[src_v.at[pl.ds(b * CB, CB)]], rows_v,
                             gsem).wait()
            pltpu.sync_copy(eac.at[pl.ds(base + b * CB, CB)], w_v)

            def edge_body(e, c2):
                w0 = w_v[e, pl.ds(0, 16)]
                w1 = w_v[e, pl.ds(16, 16)]
                w2 = w_v[e, pl.ds(32, 16)]
                ws = [w0, w1, w1, w1, w2, w2, w2, w2, w2]
                for p in range(9):
                    sl = pl.ds(16 * p, 16)
                    rows_v[e, sl] = rows_v[e, sl] * ws[p]
                return c2

            lax.fori_loop(0, CB, edge_body, 0)
            pltpu.sync_copy(rows_v, acc.at[dst_v.at[b]], add=True)
            return carry

        lax.fori_loop(0, NBATCH, batch_body, 0)
        plsc.subcore_barrier()
        pltpu.sync_copy(acc.at[pl.ds(sub * NT, NT)],
                        out.at[chunk, pl.ds(sub * NT, NT)])
        plsc.subcore_barrier()


_sc_call = pl.kernel(
    _sc_msg_body,
    out_type=jax.ShapeDtypeStruct((8, NPAD, F), jnp.float32),
    mesh=plsc.VectorSubcoreMesh(core_axis_name="c", subcore_axis_name="s"),
    compiler_params=pltpu.CompilerParams(use_tc_tiling_on_sc=False),
    scratch_types=[
        pltpu.VMEM_SHARED((NPAD, F), jnp.float32),
        pltpu.VMEM((ET,), jnp.int32),
        pltpu.VMEM((NBATCH, CB), jnp.int32),
        pltpu.VMEM((CB, F), jnp.float32),
        pltpu.VMEM((CB, 48), jnp.float32),
        pltpu.SemaphoreType.DMA,
    ],
)


def _ws3_perm():
    perm = np.empty((3 * H,), dtype=np.int32)
    for c in range(H // 16):
        for g in range(3):
            for hh in range(16):
                perm[c * 48 + g * 16 + hh] = 3 * (16 * c + hh) + g
    return perm


_PERM = _ws3_perm()


def kernel(X, edge_index, edge_weight, edge_attr, q, ws1, bs1, ws2, bs2, ws3,
           bs3, wt):
    X9 = jnp.transpose(X.reshape(N, H, 9), (0, 2, 1))
    ew2 = edge_weight.reshape(E // EB, 1, EB)
    q2 = q.reshape(N // NB, 1, NB)
    src = edge_index[1]
    dst = edge_index[0]
    srcc = (src[None, :]
            + (jnp.arange(8, dtype=jnp.int32) * N)[:, None]).reshape(-1)
    dst2d = dst.reshape(16, NBATCH, CB)
    for l in range(L):
        w3p = ws3[l][_PERM]
        b3p = bs3[l][_PERM]
        ea = _mlp_call(edge_attr, ew2, ws1[l], bs1[l].reshape(1, H),
                       ws2[l], bs2[l].reshape(1, 2 * H), w3p,
                       b3p.reshape(1, 3 * H))
        Xn9, feat9, Y9 = _prep_call(X9, wt[l, 0], wt[l, 1], wt[l, 2])
        featc = jnp.transpose(feat9.reshape(N, 9, 8, 16),
                              (2, 0, 1, 3)).reshape(8 * N, F)
        eac = jnp.transpose(ea.reshape(E, 8, 48), (1, 0, 2)).reshape(8 * E, 48)
        msgc = _sc_call(featc, eac, srcc, dst2d)[:, :N]
        msgc9 = jnp.transpose(msgc.reshape(8, N, 9, 16),
                              (1, 2, 0, 3)).reshape(N, 9, H)
        X9 = _post_call(msgc9, Y9, Xn9, q2, wt[l, 3], wt[l, 4], wt[l, 5])
    return jnp.transpose(X9, (0, 2, 1)).reshape(N, H, 3, 3)


# speedup vs baseline: 19.8165x; 1.1749x over previous
import functools
import math

import jax
import jax.numpy as jnp
import numpy as np
from jax import lax
from jax.experimental import pallas as pl
from jax.experimental.pallas import tpu as pltpu
from jax.experimental.pallas import tpu_sc as plsc

N = 10000
E = 160000
H = 128
R = 32
L = 2
CUTOFF_UPPER = 4.5

EB = 2000
NB = 400


def _silu(x):
    return x / (1.0 + jnp.exp(-x))


def _mlp_body(attr_ref, ew_ref, w1_ref, b1_ref, w2_ref, b2_ref, w3_ref, b3_ref,
              out_ref):
    x = _silu(jnp.dot(attr_ref[...], w1_ref[...].T,
                      preferred_element_type=jnp.float32) + b1_ref[...])
    x = _silu(jnp.dot(x, w2_ref[...].T,
                      preferred_element_type=jnp.float32) + b2_ref[...])
    x = _silu(jnp.dot(x, w3_ref[...].T,
                      preferred_element_type=jnp.float32) + b3_ref[...])
    d = ew_ref[...].reshape(-1, 1)
    c = 0.5 * (jnp.cos(d * (math.pi / CUTOFF_UPPER)) + 1.0)
    c = jnp.where(d < CUTOFF_UPPER, c, 0.0)
    x = x * c
    out_ref[...] = jnp.transpose(x.reshape(EB, 8, 48), (1, 0, 2))


def _mlp_call(edge_attr, ew2, w1, b1, w2, b2, w3, b3):
    grid = (E // EB,)
    return pl.pallas_call(
        _mlp_body,
        grid=grid,
        in_specs=[
            pl.BlockSpec((EB, R), lambda i: (i, 0)),
            pl.BlockSpec((1, 1, EB), lambda i: (i, 0, 0)),
            pl.BlockSpec((H, R), lambda i: (0, 0)),
            pl.BlockSpec((1, H), lambda i: (0, 0)),
            pl.BlockSpec((2 * H, H), lambda i: (0, 0)),
            pl.BlockSpec((1, 2 * H), lambda i: (0, 0)),
            pl.BlockSpec((3 * H, 2 * H), lambda i: (0, 0)),
            pl.BlockSpec((1, 3 * H), lambda i: (0, 0)),
        ],
        out_specs=pl.BlockSpec((8, EB, 48), lambda i: (0, i, 0)),
        out_shape=jax.ShapeDtypeStruct((8, E, 48), jnp.float32),
    )(edge_attr, ew2, w1, b1, w2, b2, w3, b3)


def _decompose(t):
    trm = (t[:, 0] + t[:, 4] + t[:, 8]) * (1.0 / 3.0)
    a01 = 0.5 * (t[:, 1] - t[:, 3])
    a02 = 0.5 * (t[:, 2] - t[:, 6])
    a12 = 0.5 * (t[:, 5] - t[:, 7])
    s00 = t[:, 0] - trm
    s01 = 0.5 * (t[:, 1] + t[:, 3])
    s02 = 0.5 * (t[:, 2] + t[:, 6])
    s11 = t[:, 4] - trm
    s12 = 0.5 * (t[:, 5] + t[:, 7])
    return [trm, a01, a02, a12, s00, s01, s02, s11, s12]


def _mix(c, w0, w1, w2):
    out = [jnp.dot(c[0], w0.T, preferred_element_type=jnp.float32)]
    for k in range(1, 4):
        out.append(jnp.dot(c[k], w1.T, preferred_element_type=jnp.float32))
    for k in range(4, 9):
        out.append(jnp.dot(c[k], w2.T, preferred_element_type=jnp.float32))
    return out


def _recon(c):
    i, a01, a02, a12, s00, s01, s02, s11, s12 = c
    return [i + s00, s01 + a01, s02 + a02,
            s01 - a01, i + s11, s12 + a12,
            s02 - a02, s12 - a12, i - s00 - s11]


def _mm33(a, b):
    out = []
    for i in range(3):
        for j in range(3):
            acc = a[3 * i] * b[j]
            for k in range(1, 3):
                acc = acc + a[3 * i + k] * b[3 * k + j]
            out.append(acc)
    return out


def _prep_body(x_ref, w0_ref, w1_ref, w2_ref, xn_ref, feat_ref, y_ref):
    x = x_ref[...]
    norm = jnp.sum(x * x, axis=1)
    xn = x * (1.0 / (norm + 1.0))[:, None, :]
    xn_ref[...] = xn
    c0 = _decompose(xn)
    feat = _mix(c0, w0_ref[...], w1_ref[...], w2_ref[...])
    f9 = jnp.stack(feat, axis=1)
    feat_ref[...] = jnp.transpose(f9.reshape(NB, 9, 8, 16),
                                  (2, 0, 1, 3)).reshape(8, NB, F)
    y_ref[...] = jnp.stack(_recon(feat), axis=1)


def _prep_call(X9, w0, w1, w2):
    grid = (N // NB,)
    outs = [jax.ShapeDtypeStruct((N, 9, H), jnp.float32),
            jax.ShapeDtypeStruct((8, N, F), jnp.float32),
            jax.ShapeDtypeStruct((N, 9, H), jnp.float32)]
    return pl.pallas_call(
        _prep_body,
        grid=grid,
        in_specs=[
            pl.BlockSpec((NB, 9, H), lambda i: (i, 0, 0)),
            pl.BlockSpec((H, H), lambda i: (0, 0)),
            pl.BlockSpec((H, H), lambda i: (0, 0)),
            pl.BlockSpec((H, H), lambda i: (0, 0)),
        ],
        out_specs=[pl.BlockSpec((NB, 9, H), lambda i: (i, 0, 0)),
                   pl.BlockSpec((8, NB, F), lambda i: (0, i, 0)),
                   pl.BlockSpec((NB, 9, H), lambda i: (i, 0, 0))],
        out_shape=outs,
    )(X9, w0, w1, w2)


def _post_body(m_ref, y_ref, xn_ref, q_ref, w3_ref, w4_ref, w5_ref, out_ref):
    m9 = jnp.transpose(m_ref[...].reshape(8, NB, 9, 16),
                       (1, 2, 0, 3)).reshape(NB, 9, H)
    mc = [m9[:, k] for k in range(9)]
    m = _recon(mc)
    y = [y_ref[:, k] for k in range(9)]
    a2 = _mm33(m, y)
    b = _mm33(y, m)
    fac = 1.0 + 0.1 * q_ref[...].reshape(-1, 1)
    t = [fac * (a2[k] + b[k]) for k in range(9)]
    c2 = _decompose(jnp.stack(t, axis=1))
    dX = _recon(_mix(c2, w3_ref[...], w4_ref[...], w5_ref[...]))
    dd = _mm33(dX, dX)
    xn = xn_ref[...]
    out = [xn[:, k] + dX[k] + dd[k] for k in range(9)]
    out_ref[...] = jnp.stack(out, axis=1)


def _post_call(msgc, Y9, Xn9, q2, w3, w4, w5):
    grid = (N // NB,)
    return pl.pallas_call(
        _post_body,
        grid=grid,
        in_specs=[
            pl.BlockSpec((8, NB, F), lambda i: (0, i, 0)),
            pl.BlockSpec((NB, 9, H), lambda i: (i, 0, 0)),
            pl.BlockSpec((NB, 9, H), lambda i: (i, 0, 0)),
            pl.BlockSpec((1, 1, NB), lambda i: (i, 0, 0)),
            pl.BlockSpec((H, H), lambda i: (0, 0)),
            pl.BlockSpec((H, H), lambda i: (0, 0)),
            pl.BlockSpec((H, H), lambda i: (0, 0)),
        ],
        out_specs=pl.BlockSpec((NB, 9, H), lambda i: (i, 0, 0)),
        out_shape=jax.ShapeDtypeStruct((N, 9, H), jnp.float32),
    )(msgc, Y9, Xn9, q2, w3, w4, w5)


CB = 80
ET = E // 16
NBATCH = ET // CB
NPAD = 10240
NT = NPAD // 16
ZR = NT // 5
F = 144


def _sc_msg_body(featc, eac, srcc, dst2d, out, acc, src_v, dst_v,
                 rows_v, w_v, gsem):
    core = lax.axis_index("c")
    sub = lax.axis_index("s")
    z16 = jnp.zeros((16,), jnp.float32)
    pltpu.sync_copy(dst2d.at[sub], dst_v)

    for cc in range(4):
        chunk = core * 4 + cc

        def zrow(i, carry):
            for p in range(9):
                rows_v[i, pl.ds(16 * p, 16)] = z16
            return carry

        lax.fori_loop(0, CB, zrow, 0)
        for z in range(NT // CB):
            pltpu.sync_copy(rows_v, acc.at[pl.ds(sub * NT + z * CB, CB)])
        plsc.subcore_barrier()
        base = chunk * E + sub * ET
        pltpu.sync_copy(srcc.at[pl.ds(base, ET)], src_v)

        def batch_body(b, carry):
            pltpu.async_copy(featc.at[src_v.at[pl.ds(b * CB, CB)]], rows_v,
                             gsem).wait()
            pltpu.sync_copy(eac.at[pl.ds(base + b * CB, CB)], w_v)

            def edge_body(e, c2):
                w0 = w_v[e, pl.ds(0, 16)]
                w1 = w_v[e, pl.ds(16, 16)]
                w2 = w_v[e, pl.ds(32, 16)]
                ws = [w0, w1, w1, w1, w2, w2, w2, w2, w2]
                for p in range(9):
                    sl = pl.ds(16 * p, 16)
                    rows_v[e, sl] = rows_v[e, sl] * ws[p]
                return c2

            lax.fori_loop(0, CB, edge_body, 0)
            pltpu.sync_copy(rows_v, acc.at[dst_v.at[b]], add=True)
            return carry

        lax.fori_loop(0, NBATCH, batch_body, 0)
        plsc.subcore_barrier()
        pltpu.sync_copy(acc.at[pl.ds(sub * NT, NT)],
                        out.at[chunk, pl.ds(sub * NT, NT)])
        plsc.subcore_barrier()


_sc_call = pl.kernel(
    _sc_msg_body,
    out_type=jax.ShapeDtypeStruct((8, NPAD, F), jnp.float32),
    mesh=plsc.VectorSubcoreMesh(core_axis_name="c", subcore_axis_name="s"),
    compiler_params=pltpu.CompilerParams(use_tc_tiling_on_sc=False),
    scratch_types=[
        pltpu.VMEM_SHARED((NPAD, F), jnp.float32),
        pltpu.VMEM((ET,), jnp.int32),
        pltpu.VMEM((NBATCH, CB), jnp.int32),
        pltpu.VMEM((CB, F), jnp.float32),
        pltpu.VMEM((CB, 48), jnp.float32),
        pltpu.SemaphoreType.DMA,
    ],
)


def _ws3_perm():
    perm = np.empty((3 * H,), dtype=np.int32)
    for c in range(H // 16):
        for g in range(3):
            for hh in range(16):
                perm[c * 48 + g * 16 + hh] = 3 * (16 * c + hh) + g
    return perm


_PERM = _ws3_perm()


def kernel(X, edge_index, edge_weight, edge_attr, q, ws1, bs1, ws2, bs2, ws3,
           bs3, wt):
    X9 = jnp.transpose(X.reshape(N, H, 9), (0, 2, 1))
    ew2 = edge_weight.reshape(E // EB, 1, EB)
    q2 = q.reshape(N // NB, 1, NB)
    src = edge_index[1]
    dst = edge_index[0]
    srcc = (src[None, :]
            + (jnp.arange(8, dtype=jnp.int32) * N)[:, None]).reshape(-1)
    dst2d = dst.reshape(16, NBATCH, CB)
    for l in range(L):
        w3p = ws3[l][_PERM]
        b3p = bs3[l][_PERM]
        ea = _mlp_call(edge_attr, ew2, ws1[l], bs1[l].reshape(1, H),
                       ws2[l], bs2[l].reshape(1, 2 * H), w3p,
                       b3p.reshape(1, 3 * H))
        Xn9, featc, Y9 = _prep_call(X9, wt[l, 0], wt[l, 1], wt[l, 2])
        msgc = _sc_call(featc.reshape(8 * N, F), ea.reshape(8 * E, 48),
                        srcc, dst2d)
        X9 = _post_call(msgc, Y9, Xn9, q2, wt[l, 3], wt[l, 4], wt[l, 5])
    return jnp.transpose(X9, (0, 2, 1)).reshape(N, H, 3, 3)


# speedup vs baseline: 21.8223x; 1.1012x over previous
import functools
import math

import jax
import jax.numpy as jnp
import numpy as np
from jax import lax
from jax.experimental import pallas as pl
from jax.experimental.pallas import tpu as pltpu
from jax.experimental.pallas import tpu_sc as plsc

N = 10000
E = 160000
H = 128
R = 32
L = 2
CUTOFF_UPPER = 4.5

EB = 2000
NB = 400


def _silu(x):
    return x / (1.0 + jnp.exp(-x))


def _mlp_body(attr_ref, ew_ref, w1_ref, b1_ref, w2_ref, b2_ref, w3_ref, b3_ref,
              out_ref):
    x = _silu(jnp.dot(attr_ref[...], w1_ref[...].T,
                      preferred_element_type=jnp.float32) + b1_ref[...])
    x = _silu(jnp.dot(x, w2_ref[...].T,
                      preferred_element_type=jnp.float32) + b2_ref[...])
    x = _silu(jnp.dot(x, w3_ref[...].T,
                      preferred_element_type=jnp.float32) + b3_ref[...])
    d = ew_ref[...].reshape(-1, 1)
    c = 0.5 * (jnp.cos(d * (math.pi / CUTOFF_UPPER)) + 1.0)
    c = jnp.where(d < CUTOFF_UPPER, c, 0.0)
    x = x * c
    out_ref[...] = jnp.transpose(x.reshape(EB, 8, 48), (1, 0, 2))


def _mlp_call(edge_attr, ew2, w1, b1, w2, b2, w3, b3):
    grid = (E // EB,)
    return pl.pallas_call(
        _mlp_body,
        grid=grid,
        in_specs=[
            pl.BlockSpec((EB, R), lambda i: (i, 0)),
            pl.BlockSpec((1, 1, EB), lambda i: (i, 0, 0)),
            pl.BlockSpec((H, R), lambda i: (0, 0)),
            pl.BlockSpec((1, H), lambda i: (0, 0)),
            pl.BlockSpec((2 * H, H), lambda i: (0, 0)),
            pl.BlockSpec((1, 2 * H), lambda i: (0, 0)),
            pl.BlockSpec((3 * H, 2 * H), lambda i: (0, 0)),
            pl.BlockSpec((1, 3 * H), lambda i: (0, 0)),
        ],
        out_specs=pl.BlockSpec((8, EB, 48), lambda i: (0, i, 0)),
        out_shape=jax.ShapeDtypeStruct((8, E, 48), jnp.float32),
    )(edge_attr, ew2, w1, b1, w2, b2, w3, b3)


def _decompose(t):
    trm = (t[:, 0] + t[:, 4] + t[:, 8]) * (1.0 / 3.0)
    a01 = 0.5 * (t[:, 1] - t[:, 3])
    a02 = 0.5 * (t[:, 2] - t[:, 6])
    a12 = 0.5 * (t[:, 5] - t[:, 7])
    s00 = t[:, 0] - trm
    s01 = 0.5 * (t[:, 1] + t[:, 3])
    s02 = 0.5 * (t[:, 2] + t[:, 6])
    s11 = t[:, 4] - trm
    s12 = 0.5 * (t[:, 5] + t[:, 7])
    return [trm, a01, a02, a12, s00, s01, s02, s11, s12]


def _mix(c, w0, w1, w2):
    out = [jnp.dot(c[0], w0.T, preferred_element_type=jnp.float32)]
    for k in range(1, 4):
        out.append(jnp.dot(c[k], w1.T, preferred_element_type=jnp.float32))
    for k in range(4, 9):
        out.append(jnp.dot(c[k], w2.T, preferred_element_type=jnp.float32))
    return out


def _recon(c):
    i, a01, a02, a12, s00, s01, s02, s11, s12 = c
    return [i + s00, s01 + a01, s02 + a02,
            s01 - a01, i + s11, s12 + a12,
            s02 - a02, s12 - a12, i - s00 - s11]


def _mm33(a, b):
    out = []
    for i in range(3):
        for j in range(3):
            acc = a[3 * i] * b[j]
            for k in range(1, 3):
                acc = acc + a[3 * i + k] * b[3 * k + j]
            out.append(acc)
    return out


def _prep_body(x_ref, w0_ref, w1_ref, w2_ref, xn_ref, feat_ref, y_ref):
    x = x_ref[...]
    norm = jnp.sum(x * x, axis=1)
    xn = x * (1.0 / (norm + 1.0))[:, None, :]
    xn_ref[...] = xn
    c0 = _decompose(xn)
    feat = _mix(c0, w0_ref[...], w1_ref[...], w2_ref[...])
    f9 = jnp.stack(feat, axis=1)
    feat_ref[...] = jnp.transpose(f9.reshape(NB, 9, 8, 16),
                                  (2, 0, 1, 3)).reshape(8, NB, F)
    y_ref[...] = jnp.stack(_recon(feat), axis=1)


def _prep_call(X9, w0, w1, w2):
    grid = (N // NB,)
    outs = [jax.ShapeDtypeStruct((N, 9, H), jnp.float32),
            jax.ShapeDtypeStruct((8, N, F), jnp.float32),
            jax.ShapeDtypeStruct((N, 9, H), jnp.float32)]
    return pl.pallas_call(
        _prep_body,
        grid=grid,
        in_specs=[
            pl.BlockSpec((NB, 9, H), lambda i: (i, 0, 0)),
            pl.BlockSpec((H, H), lambda i: (0, 0)),
            pl.BlockSpec((H, H), lambda i: (0, 0)),
            pl.BlockSpec((H, H), lambda i: (0, 0)),
        ],
        out_specs=[pl.BlockSpec((NB, 9, H), lambda i: (i, 0, 0)),
                   pl.BlockSpec((8, NB, F), lambda i: (0, i, 0)),
                   pl.BlockSpec((NB, 9, H), lambda i: (i, 0, 0))],
        out_shape=outs,
    )(X9, w0, w1, w2)


def _post_body(m_ref, y_ref, xn_ref, q_ref, w3_ref, w4_ref, w5_ref, out_ref):
    m9 = jnp.transpose(m_ref[...].reshape(8, NB, 9, 16),
                       (1, 2, 0, 3)).reshape(NB, 9, H)
    mc = [m9[:, k] for k in range(9)]
    m = _recon(mc)
    y = [y_ref[:, k] for k in range(9)]
    a2 = _mm33(m, y)
    b = _mm33(y, m)
    fac = 1.0 + 0.1 * q_ref[...].reshape(-1, 1)
    t = [fac * (a2[k] + b[k]) for k in range(9)]
    c2 = _decompose(jnp.stack(t, axis=1))
    dX = _recon(_mix(c2, w3_ref[...], w4_ref[...], w5_ref[...]))
    dd = _mm33(dX, dX)
    xn = xn_ref[...]
    out = [xn[:, k] + dX[k] + dd[k] for k in range(9)]
    out_ref[...] = jnp.stack(out, axis=1)


def _post_call(msgc, Y9, Xn9, q2, w3, w4, w5):
    grid = (N // NB,)
    return pl.pallas_call(
        _post_body,
        grid=grid,
        in_specs=[
            pl.BlockSpec((8, NB, F), lambda i: (0, i, 0)),
            pl.BlockSpec((NB, 9, H), lambda i: (i, 0, 0)),
            pl.BlockSpec((NB, 9, H), lambda i: (i, 0, 0)),
            pl.BlockSpec((1, 1, NB), lambda i: (i, 0, 0)),
            pl.BlockSpec((H, H), lambda i: (0, 0)),
            pl.BlockSpec((H, H), lambda i: (0, 0)),
            pl.BlockSpec((H, H), lambda i: (0, 0)),
        ],
        out_specs=pl.BlockSpec((NB, 9, H), lambda i: (i, 0, 0)),
        out_shape=jax.ShapeDtypeStruct((N, 9, H), jnp.float32),
    )(msgc, Y9, Xn9, q2, w3, w4, w5)


CB = 40
ET = E // 16
NBATCH = ET // CB
NPAD = 10112
NT = NPAD // 16
F = 144


def _sc_msg_body(featc, eac, srcc, dst2d, out, acc, src_v, dst_v,
                 rows0_v, rows1_v, w0_v, w1_v, gsem0, gsem1, wsem0, wsem1):
    core = lax.axis_index("c")
    sub = lax.axis_index("s")
    z16 = jnp.zeros((16,), jnp.float32)
    pltpu.sync_copy(dst2d.at[sub], dst_v)

    def _make_edge_body(rows_v, w_v):
        def edge_body(e, c2):
            w0 = w_v[e, pl.ds(0, 16)]
            w1 = w_v[e, pl.ds(16, 16)]
            w2 = w_v[e, pl.ds(32, 16)]
            ws = [w0, w1, w1, w1, w2, w2, w2, w2, w2]
            for p in range(9):
                sl = pl.ds(16 * p, 16)
                rows_v[e, sl] = rows_v[e, sl] * ws[p]
            return c2
        return edge_body

    for cc in range(4):
        chunk = core * 4 + cc

        def zrow(i, carry):
            for p in range(9):
                rows0_v[i, pl.ds(16 * p, 16)] = z16
            return carry

        lax.fori_loop(0, CB, zrow, 0)
        off = 0
        while off < NT:
            n = min(CB, NT - off)
            pltpu.sync_copy(rows0_v.at[pl.ds(0, n)],
                            acc.at[pl.ds(sub * NT + off, n)])
            off += n
        plsc.subcore_barrier()
        base = chunk * E + sub * ET
        pltpu.sync_copy(srcc.at[pl.ds(base, ET)], src_v)

        def pair_body(k, carry):
            b0 = 2 * k
            b1 = b0 + 1
            g0 = pltpu.async_copy(
                featc.at[src_v.at[pl.ds(b0 * CB, CB)]], rows0_v, gsem0)
            g1 = pltpu.async_copy(
                featc.at[src_v.at[pl.ds(b1 * CB, CB)]], rows1_v, gsem1)
            h0 = pltpu.async_copy(
                eac.at[pl.ds(base + b0 * CB, CB)], w0_v, wsem0)
            h1 = pltpu.async_copy(
                eac.at[pl.ds(base + b1 * CB, CB)], w1_v, wsem1)
            g0.wait()
            h0.wait()
            lax.fori_loop(0, CB, _make_edge_body(rows0_v, w0_v), 0)
            pltpu.sync_copy(rows0_v, acc.at[dst_v.at[b0]], add=True)
            g1.wait()
            h1.wait()
            lax.fori_loop(0, CB, _make_edge_body(rows1_v, w1_v), 0)
            pltpu.sync_copy(rows1_v, acc.at[dst_v.at[b1]], add=True)
            return carry

        lax.fori_loop(0, NBATCH // 2, pair_body, 0)
        plsc.subcore_barrier()
        pltpu.sync_copy(acc.at[pl.ds(sub * NT, NT)],
                        out.at[chunk, pl.ds(sub * NT, NT)])
        plsc.subcore_barrier()


_sc_call = pl.kernel(
    _sc_msg_body,
    out_type=jax.ShapeDtypeStruct((8, NPAD, F), jnp.float32),
    mesh=plsc.VectorSubcoreMesh(core_axis_name="c", subcore_axis_name="s"),
    compiler_params=pltpu.CompilerParams(use_tc_tiling_on_sc=False),
    scratch_types=[
        pltpu.VMEM_SHARED((NPAD, F), jnp.float32),
        pltpu.VMEM((ET,), jnp.int32),
        pltpu.VMEM((NBATCH, CB), jnp.int32),
        pltpu.VMEM((CB, F), jnp.float32),
        pltpu.VMEM((CB, F), jnp.float32),
        pltpu.VMEM((CB, 48), jnp.float32),
        pltpu.VMEM((CB, 48), jnp.float32),
        pltpu.SemaphoreType.DMA,
        pltpu.SemaphoreType.DMA,
        pltpu.SemaphoreType.DMA,
        pltpu.SemaphoreType.DMA,
    ],
)


def _ws3_perm():
    perm = np.empty((3 * H,), dtype=np.int32)
    for c in range(H // 16):
        for g in range(3):
            for hh in range(16):
                perm[c * 48 + g * 16 + hh] = 3 * (16 * c + hh) + g
    return perm


_PERM = _ws3_perm()


def kernel(X, edge_index, edge_weight, edge_attr, q, ws1, bs1, ws2, bs2, ws3,
           bs3, wt):
    X9 = jnp.transpose(X.reshape(N, H, 9), (0, 2, 1))
    ew2 = edge_weight.reshape(E // EB, 1, EB)
    q2 = q.reshape(N // NB, 1, NB)
    src = edge_index[1]
    dst = edge_index[0]
    srcc = (src[None, :]
            + (jnp.arange(8, dtype=jnp.int32) * N)[:, None]).reshape(-1)
    dst2d = dst.reshape(16, NBATCH, CB)
    for l in range(L):
        w3p = ws3[l][_PERM]
        b3p = bs3[l][_PERM]
        ea = _mlp_call(edge_attr, ew2, ws1[l], bs1[l].reshape(1, H),
                       ws2[l], bs2[l].reshape(1, 2 * H), w3p,
                       b3p.reshape(1, 3 * H))
        Xn9, featc, Y9 = _prep_call(X9, wt[l, 0], wt[l, 1], wt[l, 2])
        msgc = _sc_call(featc.reshape(8 * N, F), ea.reshape(8 * E, 48),
                        srcc, dst2d)
        X9 = _post_call(msgc, Y9, Xn9, q2, wt[l, 3], wt[l, 4], wt[l, 5])
    return jnp.transpose(X9, (0, 2, 1)).reshape(N, H, 3, 3)


# speedup vs baseline: 22.5694x; 1.0342x over previous
import functools
import math

import jax
import jax.numpy as jnp
import numpy as np
from jax import lax
from jax.experimental import pallas as pl
from jax.experimental.pallas import tpu as pltpu
from jax.experimental.pallas import tpu_sc as plsc

N = 10000
E = 160000
H = 128
R = 32
L = 2
CUTOFF_UPPER = 4.5

EB = 2000
NB = 400


def _silu(x):
    return x / (1.0 + jnp.exp(-x))


def _mlp_body(attr_ref, ew_ref, w1_ref, b1_ref, w2_ref, b2_ref, w3_ref, b3_ref,
              out_ref):
    x = _silu(jnp.dot(attr_ref[...], w1_ref[...].T,
                      preferred_element_type=jnp.float32) + b1_ref[...])
    x = _silu(jnp.dot(x, w2_ref[...].T,
                      preferred_element_type=jnp.float32) + b2_ref[...])
    x = _silu(jnp.dot(x, w3_ref[...].T,
                      preferred_element_type=jnp.float32) + b3_ref[...])
    d = ew_ref[...].reshape(-1, 1)
    c = 0.5 * (jnp.cos(d * (math.pi / CUTOFF_UPPER)) + 1.0)
    c = jnp.where(d < CUTOFF_UPPER, c, 0.0)
    x = x * c
    out_ref[...] = jnp.transpose(x.reshape(EB, 8, 48), (1, 0, 2))


def _mlp_call(edge_attr, ew2, w1, b1, w2, b2, w3, b3):
    grid = (E // EB,)
    return pl.pallas_call(
        _mlp_body,
        grid=grid,
        in_specs=[
            pl.BlockSpec((EB, R), lambda i: (i, 0)),
            pl.BlockSpec((1, 1, EB), lambda i: (i, 0, 0)),
            pl.BlockSpec((H, R), lambda i: (0, 0)),
            pl.BlockSpec((1, H), lambda i: (0, 0)),
            pl.BlockSpec((2 * H, H), lambda i: (0, 0)),
            pl.BlockSpec((1, 2 * H), lambda i: (0, 0)),
            pl.BlockSpec((3 * H, 2 * H), lambda i: (0, 0)),
            pl.BlockSpec((1, 3 * H), lambda i: (0, 0)),
        ],
        out_specs=pl.BlockSpec((8, EB, 48), lambda i: (0, i, 0)),
        out_shape=jax.ShapeDtypeStruct((8, E, 48), jnp.float32),
    )(edge_attr, ew2, w1, b1, w2, b2, w3, b3)


def _decompose(t):
    trm = (t[:, 0] + t[:, 4] + t[:, 8]) * (1.0 / 3.0)
    a01 = 0.5 * (t[:, 1] - t[:, 3])
    a02 = 0.5 * (t[:, 2] - t[:, 6])
    a12 = 0.5 * (t[:, 5] - t[:, 7])
    s00 = t[:, 0] - trm
    s01 = 0.5 * (t[:, 1] + t[:, 3])
    s02 = 0.5 * (t[:, 2] + t[:, 6])
    s11 = t[:, 4] - trm
    s12 = 0.5 * (t[:, 5] + t[:, 7])
    return [trm, a01, a02, a12, s00, s01, s02, s11, s12]


def _mix(c, w0, w1, w2):
    out = [jnp.dot(c[0], w0.T, preferred_element_type=jnp.float32)]
    for k in range(1, 4):
        out.append(jnp.dot(c[k], w1.T, preferred_element_type=jnp.float32))
    for k in range(4, 9):
        out.append(jnp.dot(c[k], w2.T, preferred_element_type=jnp.float32))
    return out


def _recon(c):
    i, a01, a02, a12, s00, s01, s02, s11, s12 = c
    return [i + s00, s01 + a01, s02 + a02,
            s01 - a01, i + s11, s12 + a12,
            s02 - a02, s12 - a12, i - s00 - s11]


def _mm33(a, b):
    out = []
    for i in range(3):
        for j in range(3):
            acc = a[3 * i] * b[j]
            for k in range(1, 3):
                acc = acc + a[3 * i + k] * b[3 * k + j]
            out.append(acc)
    return out


def _prep_body(x_ref, w0_ref, w1_ref, w2_ref, xn_ref, feat_ref, y_ref):
    x = x_ref[...]
    norm = jnp.sum(x * x, axis=1)
    xn = x * (1.0 / (norm + 1.0))[:, None, :]
    xn_ref[...] = xn
    c0 = _decompose(xn)
    feat = _mix(c0, w0_ref[...], w1_ref[...], w2_ref[...])
    f9 = jnp.stack(feat, axis=1)
    feat_ref[...] = jnp.transpose(f9.reshape(NB, 9, 8, 16),
                                  (2, 0, 1, 3)).reshape(8, NB, F)
    y_ref[...] = jnp.stack(_recon(feat), axis=1)


def _prep_call(X9, w0, w1, w2):
    grid = (N // NB,)
    outs = [jax.ShapeDtypeStruct((N, 9, H), jnp.float32),
            jax.ShapeDtypeStruct((8, N, F), jnp.float32),
            jax.ShapeDtypeStruct((N, 9, H), jnp.float32)]
    return pl.pallas_call(
        _prep_body,
        grid=grid,
        in_specs=[
            pl.BlockSpec((NB, 9, H), lambda i: (i, 0, 0)),
            pl.BlockSpec((H, H), lambda i: (0, 0)),
            pl.BlockSpec((H, H), lambda i: (0, 0)),
            pl.BlockSpec((H, H), lambda i: (0, 0)),
        ],
        out_specs=[pl.BlockSpec((NB, 9, H), lambda i: (i, 0, 0)),
                   pl.BlockSpec((8, NB, F), lambda i: (0, i, 0)),
                   pl.BlockSpec((NB, 9, H), lambda i: (i, 0, 0))],
        out_shape=outs,
    )(X9, w0, w1, w2)


def _post_body(m_ref, y_ref, xn_ref, q_ref, w3_ref, w4_ref, w5_ref, out_ref):
    m9 = jnp.transpose(m_ref[...].reshape(8, NB, 9, 16),
                       (1, 2, 0, 3)).reshape(NB, 9, H)
    mc = [m9[:, k] for k in range(9)]
    m = _recon(mc)
    y = [y_ref[:, k] for k in range(9)]
    a2 = _mm33(m, y)
    b = _mm33(y, m)
    fac = 1.0 + 0.1 * q_ref[...].reshape(-1, 1)
    t = [fac * (a2[k] + b[k]) for k in range(9)]
    c2 = _decompose(jnp.stack(t, axis=1))
    dX = _recon(_mix(c2, w3_ref[...], w4_ref[...], w5_ref[...]))
    dd = _mm33(dX, dX)
    xn = xn_ref[...]
    out = [xn[:, k] + dX[k] + dd[k] for k in range(9)]
    out_ref[...] = jnp.stack(out, axis=1)


def _post_call(msgc, Y9, Xn9, q2, w3, w4, w5):
    grid = (N // NB,)
    return pl.pallas_call(
        _post_body,
        grid=grid,
        in_specs=[
            pl.BlockSpec((8, NB, F), lambda i: (0, i, 0)),
            pl.BlockSpec((NB, 9, H), lambda i: (i, 0, 0)),
            pl.BlockSpec((NB, 9, H), lambda i: (i, 0, 0)),
            pl.BlockSpec((1, 1, NB), lambda i: (i, 0, 0)),
            pl.BlockSpec((H, H), lambda i: (0, 0)),
            pl.BlockSpec((H, H), lambda i: (0, 0)),
            pl.BlockSpec((H, H), lambda i: (0, 0)),
        ],
        out_specs=pl.BlockSpec((NB, 9, H), lambda i: (i, 0, 0)),
        out_shape=jax.ShapeDtypeStruct((N, 9, H), jnp.float32),
    )(msgc, Y9, Xn9, q2, w3, w4, w5)


CB = 40
ET = E // 16
NBATCH = ET // CB
NPAD = 10112
NT = NPAD // 16
F = 144


def _sc_msg_body(featc, eac, srcc, dst2d, out, acc, src_v, dst_v,
                 rows0_v, rows1_v, w0_v, w1_v, gsem0, gsem1, wsem0, wsem1,
                 ssem0):
    core = lax.axis_index("c")
    sub = lax.axis_index("s")
    z16 = jnp.zeros((16,), jnp.float32)
    pltpu.sync_copy(dst2d.at[sub], dst_v)

    def _make_edge_body(rows_v, w_v):
        def edge_body(e, c2):
            w0 = w_v[e, pl.ds(0, 16)]
            w1 = w_v[e, pl.ds(16, 16)]
            w2 = w_v[e, pl.ds(32, 16)]
            ws = [w0, w1, w1, w1, w2, w2, w2, w2, w2]
            for p in range(9):
                sl = pl.ds(16 * p, 16)
                rows_v[e, sl] = rows_v[e, sl] * ws[p]
            return c2
        return edge_body

    for cc in range(4):
        chunk = core * 4 + cc

        def zrow(i, carry):
            for p in range(9):
                rows0_v[i, pl.ds(16 * p, 16)] = z16
            return carry

        lax.fori_loop(0, CB, zrow, 0)
        off = 0
        while off < NT:
            n = min(CB, NT - off)
            pltpu.sync_copy(rows0_v.at[pl.ds(0, n)],
                            acc.at[pl.ds(sub * NT + off, n)])
            off += n
        plsc.subcore_barrier()
        base = chunk * E + sub * ET
        pltpu.sync_copy(srcc.at[pl.ds(base, ET)], src_v)

        def pair_body(k, carry):
            b0 = 2 * k
            b1 = b0 + 1
            g0 = pltpu.async_copy(
                featc.at[src_v.at[pl.ds(b0 * CB, CB)]], rows0_v, gsem0)
            g1 = pltpu.async_copy(
                featc.at[src_v.at[pl.ds(b1 * CB, CB)]], rows1_v, gsem1)
            h0 = pltpu.async_copy(
                eac.at[pl.ds(base + b0 * CB, CB)], w0_v, wsem0)
            h1 = pltpu.async_copy(
                eac.at[pl.ds(base + b1 * CB, CB)], w1_v, wsem1)
            g0.wait()
            h0.wait()
            lax.fori_loop(0, CB, _make_edge_body(rows0_v, w0_v), 0)
            s0 = pltpu.async_copy(rows0_v, acc.at[dst_v.at[b0]], ssem0,
                                  add=True)
            g1.wait()
            h1.wait()
            lax.fori_loop(0, CB, _make_edge_body(rows1_v, w1_v), 0)
            s0.wait()
            pltpu.sync_copy(rows1_v, acc.at[dst_v.at[b1]], add=True)
            return carry

        lax.fori_loop(0, NBATCH // 2, pair_body, 0)
        plsc.subcore_barrier()
        pltpu.sync_copy(acc.at[pl.ds(sub * NT, NT)],
                        out.at[chunk, pl.ds(sub * NT, NT)])
        plsc.subcore_barrier()


_sc_call = pl.kernel(
    _sc_msg_body,
    out_type=jax.ShapeDtypeStruct((8, NPAD, F), jnp.float32),
    mesh=plsc.VectorSubcoreMesh(core_axis_name="c", subcore_axis_name="s"),
    compiler_params=pltpu.CompilerParams(use_tc_tiling_on_sc=False),
    scratch_types=[
        pltpu.VMEM_SHARED((NPAD, F), jnp.float32),
        pltpu.VMEM((ET,), jnp.int32),
        pltpu.VMEM((NBATCH, CB), jnp.int32),
        pltpu.VMEM((CB, F), jnp.float32),
        pltpu.VMEM((CB, F), jnp.float32),
        pltpu.VMEM((CB, 48), jnp.float32),
        pltpu.VMEM((CB, 48), jnp.float32),
        pltpu.SemaphoreType.DMA,
        pltpu.SemaphoreType.DMA,
        pltpu.SemaphoreType.DMA,
        pltpu.SemaphoreType.DMA,
        pltpu.SemaphoreType.DMA,
    ],
)


def _ws3_perm():
    perm = np.empty((3 * H,), dtype=np.int32)
    for c in range(H // 16):
        for g in range(3):
            for hh in range(16):
                perm[c * 48 + g * 16 + hh] = 3 * (16 * c + hh) + g
    return perm


_PERM = _ws3_perm()


def kernel(X, edge_index, edge_weight, edge_attr, q, ws1, bs1, ws2, bs2, ws3,
           bs3, wt):
    X9 = jnp.transpose(X.reshape(N, H, 9), (0, 2, 1))
    ew2 = edge_weight.reshape(E // EB, 1, EB)
    q2 = q.reshape(N // NB, 1, NB)
    src = edge_index[1]
    dst = edge_index[0]
    srcc = (src[None, :]
            + (jnp.arange(8, dtype=jnp.int32) * N)[:, None]).reshape(-1)
    dst2d = dst.reshape(16, NBATCH, CB)
    for l in range(L):
        w3p = ws3[l][_PERM]
        b3p = bs3[l][_PERM]
        ea = _mlp_call(edge_attr, ew2, ws1[l], bs1[l].reshape(1, H),
                       ws2[l], bs2[l].reshape(1, 2 * H), w3p,
                       b3p.reshape(1, 3 * H))
        Xn9, featc, Y9 = _prep_call(X9, wt[l, 0], wt[l, 1], wt[l, 2])
        msgc = _sc_call(featc.reshape(8 * N, F), ea.reshape(8 * E, 48),
                        srcc, dst2d)
        X9 = _post_call(msgc, Y9, Xn9, q2, wt[l, 3], wt[l, 4], wt[l, 5])
    return jnp.transpose(X9, (0, 2, 1)).reshape(N, H, 3, 3)
